# Initial kernel scaffold; baseline (speedup 1.0000x reference)
import jax
import jax.numpy as jnp
from jax.experimental import pallas as pl


def kernel(state, policy):
    raise NotImplementedError("write your pallas kernel here")


# speedup vs baseline: 5.2101x; 5.2101x over previous
import functools

import jax
import jax.numpy as jnp
from jax import lax
from jax.experimental import pallas as pl
from jax.experimental.pallas import tpu as pltpu
from jax.experimental.pallas import tpu_sc as plsc


def _softmax_body(p_ref, o_ref):
    x = p_ref[...]
    m = jnp.max(x, axis=-1, keepdims=True)
    e = jnp.exp(x - m)
    o_ref[...] = e / jnp.sum(e, axis=-1, keepdims=True)


def _softmax_table(policy):
    return pl.pallas_call(
        _softmax_body,
        out_shape=jax.ShapeDtypeStruct(policy.shape, jnp.float32),
    )(policy)


_INFO = plsc.get_sparse_core_info()
_NC, _NS = _INFO.num_cores, _INFO.num_subcores
_NW = _NC * _NS

_B = 16384 * 200
_IW = 128
_NIDX_ROWS = _B // _IW
_ROWS_PER_TILE = _NIDX_ROWS // _NW
_G = 16
_NCHUNK = _ROWS_PER_TILE // _G
_D = 6


def _gather_body(idx_hbm, tab_hbm, out_hbm, idx_v, rows_v, sem):
    wid = lax.axis_index("s") * _NC + lax.axis_index("c")
    tile_row0 = wid * _ROWS_PER_TILE

    def chunk(ch, carry):
        row0 = tile_row0 + ch * _G
        pltpu.sync_copy(idx_hbm.at[pl.ds(row0, _G)], idx_v)
        copies = []
        for g in range(_G):
            copies.append(
                pltpu.async_copy(
                    tab_hbm.at[idx_v.at[g]],
                    rows_v.at[pl.ds(g * _IW, _IW)],
                    sem,
                )
            )
        for cp in copies:
            cp.wait()
        pltpu.sync_copy(rows_v, out_hbm.at[pl.ds(row0 * _IW, _G * _IW)])
        return carry

    lax.fori_loop(0, _NCHUNK, chunk, 0)


@functools.partial(jax.jit, static_argnames=())
def _sc_gather(idx2d, tab):
    mesh = plsc.VectorSubcoreMesh(core_axis_name="c", subcore_axis_name="s")
    kern = functools.partial(
        pl.kernel,
        mesh=mesh,
        out_type=jax.ShapeDtypeStruct((_B, _D), jnp.float32),
        scratch_types=[
            pltpu.VMEM((_G, _IW), jnp.int32),
            pltpu.VMEM((_G * _IW, _D), jnp.float32),
            pltpu.SemaphoreType.DMA,
        ],
        compiler_params=pltpu.CompilerParams(use_tc_tiling_on_sc=False),
    )(_gather_body)
    return kern(idx2d, tab)


def kernel(state, policy):
    tab = jax.nn.softmax(policy.astype(jnp.float32), axis=-1)
    idx2d = state.astype(jnp.int32).reshape(_NIDX_ROWS, _IW)
    out = _sc_gather(idx2d, tab)
    return out.reshape(state.shape[0], state.shape[1], _D)


# speedup vs baseline: 5.7262x; 1.0991x over previous
import functools

import jax
import jax.numpy as jnp
from jax import lax
from jax.experimental import pallas as pl
from jax.experimental.pallas import tpu as pltpu
from jax.experimental.pallas import tpu_sc as plsc

_INFO = plsc.get_sparse_core_info()
_NC, _NS = _INFO.num_cores, _INFO.num_subcores
_NW = _NC * _NS

_R, _C, _D = 16384, 200, 6
_V = 576
_ROWS_PER_TILE = _R // _NW
_C0 = 8
_CHUNK = _C0 * _C
_NCH = _ROWS_PER_TILE // _C0
_NG = _CHUNK // 16


def _body(state_hbm, poly_hbm, out_hbm, pt_v, tab_v, idx_v, rows_v, s_out0, s_out1):
    wid = lax.axis_index("s") * _NC + lax.axis_index("c")
    o_base = wid * _ROWS_PER_TILE

    pltpu.sync_copy(poly_hbm, pt_v)
    iota16 = lax.iota(jnp.int32, 16)
    colid = [jnp.full((16,), j, jnp.int32) for j in range(_D)]
    for g in range(_V // 16):
        sl = pl.ds(g * 16, 16)
        c = [pt_v[j, sl] for j in range(_D)]
        m = c[0]
        for j in range(1, _D):
            m = jnp.maximum(m, c[j])
        e = [jnp.exp(c[j] - m) for j in range(_D)]
        s = e[0]
        for j in range(1, _D):
            s = s + e[j]
        inv = 1.0 / s
        rows16 = iota16 + (g * 16)
        for j in range(_D):
            plsc.store_scatter(tab_v, [rows16, colid[j]], e[j] * inv)

    sems = (s_out0, s_out1)

    def pair(it, carry):
        for b in (0, 1):
            ch = it * 2 + b
            base_out = o_base + ch * _C0

            @pl.when(it >= 1)
            def _wait_prev():
                pltpu.make_async_copy(
                    rows_v.at[b], out_hbm.at[pl.ds(base_out, _C0)], sems[b]
                ).wait()

            pltpu.sync_copy(
                state_hbm.at[pl.ds(base_out * _C, _CHUNK)], idx_v
            )
            for g in range(_NG):
                ii = idx_v[pl.ds(g * 16, 16)]
                r16 = iota16 + (g * 16)
                i0 = r16 // _C
                i1 = r16 - i0 * _C
                for j in range(_D):
                    v = plsc.load_gather(tab_v, [ii, colid[j]])
                    plsc.store_scatter(rows_v.at[b], [i0, i1, colid[j]], v)
            pltpu.async_copy(
                rows_v.at[b], out_hbm.at[pl.ds(base_out, _C0)], sems[b]
            )
        return carry

    lax.fori_loop(0, _NCH // 2, pair, 0)

    for b in (0, 1):
        last = o_base + (_NCH - 2 + b) * _C0
        pltpu.make_async_copy(
            rows_v.at[b], out_hbm.at[pl.ds(last, _C0)], sems[b]
        ).wait()


def _sc_lookup(state_flat, policy_t):
    mesh = plsc.VectorSubcoreMesh(core_axis_name="c", subcore_axis_name="s")
    kern = functools.partial(
        pl.kernel,
        mesh=mesh,
        out_type=jax.ShapeDtypeStruct((_R, _C, _D), jnp.float32),
        scratch_types=[
            pltpu.VMEM((_D, _V), jnp.float32),
            pltpu.VMEM((_V, _D), jnp.float32),
            pltpu.VMEM((_CHUNK,), jnp.int32),
            pltpu.VMEM((2, _C0, _C, _D), jnp.float32),
            pltpu.SemaphoreType.DMA,
            pltpu.SemaphoreType.DMA,
        ],
        compiler_params=pltpu.CompilerParams(
            use_tc_tiling_on_sc=False, needs_layout_passes=False
        ),
    )(_body)
    return kern(state_flat, policy_t)


def kernel(state, policy):
    state_flat = state.astype(jnp.int32).reshape(_R * _C)
    policy_t = policy.astype(jnp.float32).T.reshape(_D, _V)
    return _sc_lookup(state_flat, policy_t)
